# baseline (speedup 1.0000x reference)
import functools

import jax
import jax.numpy as jnp
from jax import lax
from jax.experimental import pallas as pl
from jax.experimental.pallas import tpu as pltpu
from jax.experimental.pallas import tpu_sc as plsc

D = 128
E = 320000
NC = 2
NS = 16
L = 16
NW = NC * NS
EW = E // NW
CB = 200
NCHUNK = EW // CB
NG = (CB + L - 1) // L
OB = NG * L


def _make_decoder():
    mesh = plsc.VectorSubcoreMesh(core_axis_name="c", subcore_axis_name="s")

    @functools.partial(
        pl.kernel,
        mesh=mesh,
        compiler_params=pltpu.CompilerParams(needs_layout_passes=False),
        out_type=jax.ShapeDtypeStruct((E,), jnp.float32),
        scratch_types=[
            pltpu.VMEM((EW,), jnp.int32),
            pltpu.VMEM((EW,), jnp.int32),
            pltpu.VMEM((CB, D), jnp.float32),
            pltpu.VMEM((CB, D), jnp.float32),
            pltpu.VMEM((CB, D), jnp.float32),
            pltpu.VMEM((CB, D), jnp.float32),
            pltpu.VMEM((OB,), jnp.float32),
            pltpu.VMEM((OB,), jnp.float32),
            pltpu.SemaphoreType.DMA,
            pltpu.SemaphoreType.DMA,
            pltpu.SemaphoreType.DMA,
            pltpu.SemaphoreType.DMA,
            pltpu.SemaphoreType.DMA,
            pltpu.SemaphoreType.DMA,
        ],
    )
    def decoder(x_hbm, src_hbm, dst_hbm, out_hbm,
                sidx_v, didx_v, sr0, dr0, sr1, dr1, ob0, ob1,
                ss0, sd0, ss1, sd1, so0, so1):
        wid = lax.axis_index("s") * NC + lax.axis_index("c")
        base = wid * EW
        pltpu.sync_copy(src_hbm.at[pl.ds(base, EW)], sidx_v)
        pltpu.sync_copy(dst_hbm.at[pl.ds(base, EW)], didx_v)

        def start(i, sr, dr, ss, sd):
            off = i * CB
            pltpu.async_copy(x_hbm.at[sidx_v.at[pl.ds(off, CB)]], sr, ss)
            pltpu.async_copy(x_hbm.at[didx_v.at[pl.ds(off, CB)]], dr, sd)

        def wait_rows(sr, dr, ss, sd):
            pltpu.make_async_copy(x_hbm.at[pl.ds(0, CB)], sr, ss).wait()
            pltpu.make_async_copy(x_hbm.at[pl.ds(0, CB)], dr, sd).wait()

        def wait_out(ob, so):
            pltpu.make_async_copy(
                ob.at[pl.ds(0, CB)], out_hbm.at[pl.ds(base, CB)], so).wait()

        def compute(i, sr, dr, ob, so):
            off = i * CB

            last_lane = lax.iota(jnp.int32, L) == (L - 1)

            def edge_body(q, carry):
                for u in range(25):
                    e = q * 25 + u
                    acc = sr[e, pl.ds(0, L)] * dr[e, pl.ds(0, L)]
                    for c in range(1, D // L):
                        acc = acc + sr[e, pl.ds(c * L, L)] * dr[e, pl.ds(c * L, L)]
                    tot = plsc.cumsum(acc)
                    eidx = jnp.full((L,), 0, jnp.int32) + e
                    plsc.store_scatter(ob, [eidx], tot, mask=last_lane)
                return carry

            lax.fori_loop(0, CB // 25, edge_body, 0)

            def sig_body(k, carry):
                v = ob[pl.ds(k * L, L)]
                ob[pl.ds(k * L, L)] = 1.0 / (1.0 + jnp.exp(-v)) + 1e-15
                return carry

            lax.fori_loop(0, NG, sig_body, 0)
            pltpu.async_copy(
                ob.at[pl.ds(0, CB)], out_hbm.at[pl.ds(base + off, CB)], so)

        start(0, sr0, dr0, ss0, sd0)
        start(1, sr1, dr1, ss1, sd1)

        def pair_body(j, carry):
            i0 = 2 * j
            wait_rows(sr0, dr0, ss0, sd0)

            @pl.when(j > 0)
            def _():
                wait_out(ob0, so0)

            compute(i0, sr0, dr0, ob0, so0)

            @pl.when(i0 + 2 < NCHUNK)
            def _():
                start(i0 + 2, sr0, dr0, ss0, sd0)

            wait_rows(sr1, dr1, ss1, sd1)

            @pl.when(j > 0)
            def _():
                wait_out(ob1, so1)

            compute(i0 + 1, sr1, dr1, ob1, so1)

            @pl.when(i0 + 3 < NCHUNK)
            def _():
                start(i0 + 3, sr1, dr1, ss1, sd1)

            return carry

        lax.fori_loop(0, NCHUNK // 2, pair_body, 0)
        wait_out(ob0, so0)
        wait_out(ob1, so1)

    return decoder


_decoder = _make_decoder()


@jax.jit
def kernel(x, edge_index):
    ei32 = edge_index.astype(jnp.int32)
    adj_pred = _decoder(x, ei32[0], ei32[1])
    return (adj_pred, edge_index)


# speedup vs baseline: 1.0085x; 1.0085x over previous
import functools

import jax
import jax.numpy as jnp
from jax import lax
from jax.experimental import pallas as pl
from jax.experimental.pallas import tpu as pltpu
from jax.experimental.pallas import tpu_sc as plsc

D = 128
E = 320000
NC = 2
NS = 16
L = 16
NW = NC * NS
EW = E // NW
CB = 200
NCHUNK = EW // CB
NG = (CB + L - 1) // L
OB = NG * L


def _make_decoder():
    mesh = plsc.VectorSubcoreMesh(core_axis_name="c", subcore_axis_name="s")

    @functools.partial(
        pl.kernel,
        mesh=mesh,
        compiler_params=pltpu.CompilerParams(needs_layout_passes=False),
        out_type=jax.ShapeDtypeStruct((E,), jnp.float32),
        scratch_types=[
            pltpu.VMEM((EW,), jnp.int32),
            pltpu.VMEM((EW,), jnp.int32),
            pltpu.VMEM((CB, D), jnp.float32),
            pltpu.VMEM((CB, D), jnp.float32),
            pltpu.VMEM((CB, D), jnp.float32),
            pltpu.VMEM((CB, D), jnp.float32),
            pltpu.VMEM((OB,), jnp.float32),
            pltpu.VMEM((OB,), jnp.float32),
            pltpu.SemaphoreType.DMA,
            pltpu.SemaphoreType.DMA,
            pltpu.SemaphoreType.DMA,
            pltpu.SemaphoreType.DMA,
            pltpu.SemaphoreType.DMA,
            pltpu.SemaphoreType.DMA,
        ],
    )
    def decoder(x_hbm, src_hbm, dst_hbm, out_hbm,
                sidx_v, didx_v, sr0, dr0, sr1, dr1, ob0, ob1,
                ss0, sd0, ss1, sd1, so0, so1):
        wid = lax.axis_index("s") * NC + lax.axis_index("c")
        base = wid * EW
        pltpu.sync_copy(src_hbm.at[pl.ds(base, EW)], sidx_v)
        pltpu.sync_copy(dst_hbm.at[pl.ds(base, EW)], didx_v)

        def start(i, sr, dr, ss, sd):
            off = i * CB
            pltpu.async_copy(x_hbm.at[sidx_v.at[pl.ds(off, CB)]], sr, ss)
            pltpu.async_copy(x_hbm.at[didx_v.at[pl.ds(off, CB)]], dr, sd)

        def wait_rows(sr, dr, ss, sd):
            pltpu.make_async_copy(x_hbm.at[pl.ds(0, CB)], sr, ss).wait()
            pltpu.make_async_copy(x_hbm.at[pl.ds(0, CB)], dr, sd).wait()

        def wait_out(ob, so):
            pltpu.make_async_copy(
                ob.at[pl.ds(0, CB)], out_hbm.at[pl.ds(base, CB)], so).wait()

        def compute(i, sr, dr, ob, so):
            off = i * CB

            last_lane = lax.iota(jnp.int32, L) == (L - 1)

            def edge_body(q, carry):
                for u in range(8):
                    e = q * 8 + u
                    acc = sr[e, pl.ds(0, L)] * dr[e, pl.ds(0, L)]
                    for c in range(1, D // L):
                        acc = acc + sr[e, pl.ds(c * L, L)] * dr[e, pl.ds(c * L, L)]
                    tot = plsc.cumsum(acc)
                    eidx = jnp.full((L,), 0, jnp.int32) + e
                    plsc.store_scatter(ob, [eidx], tot, mask=last_lane)
                return carry

            lax.fori_loop(0, CB // 8, edge_body, 0)

            def sig_body(k, carry):
                v = ob[pl.ds(k * L, L)]
                ob[pl.ds(k * L, L)] = 1.0 / (1.0 + jnp.exp(-v)) + 1e-15
                return carry

            lax.fori_loop(0, NG, sig_body, 0)
            pltpu.async_copy(
                ob.at[pl.ds(0, CB)], out_hbm.at[pl.ds(base + off, CB)], so)

        start(0, sr0, dr0, ss0, sd0)
        start(1, sr1, dr1, ss1, sd1)

        def pair_body(j, carry):
            i0 = 2 * j
            wait_rows(sr0, dr0, ss0, sd0)

            @pl.when(j > 0)
            def _():
                wait_out(ob0, so0)

            compute(i0, sr0, dr0, ob0, so0)

            @pl.when(i0 + 2 < NCHUNK)
            def _():
                start(i0 + 2, sr0, dr0, ss0, sd0)

            wait_rows(sr1, dr1, ss1, sd1)

            @pl.when(j > 0)
            def _():
                wait_out(ob1, so1)

            compute(i0 + 1, sr1, dr1, ob1, so1)

            @pl.when(i0 + 3 < NCHUNK)
            def _():
                start(i0 + 3, sr1, dr1, ss1, sd1)

            return carry

        lax.fori_loop(0, NCHUNK // 2, pair_body, 0)
        wait_out(ob0, so0)
        wait_out(ob1, so1)

    return decoder


_decoder = _make_decoder()


@jax.jit
def kernel(x, edge_index):
    ei32 = edge_index.astype(jnp.int32)
    adj_pred = _decoder(x, ei32[0], ei32[1])
    return (adj_pred, edge_index)
